# Initial kernel scaffold; baseline (speedup 1.0000x reference)
import jax
import jax.numpy as jnp
from jax.experimental import pallas as pl


def kernel(X, W):
    raise NotImplementedError("write your pallas kernel here")


# speedup vs baseline: 1.2554x; 1.2554x over previous
import jax
import jax.numpy as jnp
from jax.experimental import pallas as pl
from jax.experimental.pallas import tpu as pltpu


def _round_up(x, m):
    return ((x + m - 1) // m) * m


def _matmul_kernel(x_ref, w_ref, o_ref):
    o_ref[...] = jnp.dot(
        x_ref[...].astype(jnp.bfloat16),
        w_ref[...],
        preferred_element_type=jnp.float32,
    ).astype(o_ref.dtype)


def kernel(X, W):
    N, D = X.shape
    D2, H = W.shape
    assert D == D2
    out_dtype = X.dtype

    Wb = W.astype(jnp.bfloat16)

    tm = min(1024, _round_up(N, 8))
    n_pad = _round_up(N, tm)
    Xp = X if n_pad == N else jnp.pad(X, ((0, n_pad - N), (0, 0)))
    grid = (n_pad // tm,)

    vmem_limit = min(
        2 * tm * D * 4 + 2 * tm * H * 4 + D * H * 2 + (4 << 20),
        64 * 1024 * 1024,
    )

    out = pl.pallas_call(
        _matmul_kernel,
        out_shape=jax.ShapeDtypeStruct((n_pad, H), out_dtype),
        grid=grid,
        in_specs=[
            pl.BlockSpec((tm, D), lambda i: (i, 0)),
            pl.BlockSpec((D, H), lambda i: (0, 0)),
        ],
        out_specs=pl.BlockSpec((tm, H), lambda i: (i, 0)),
        compiler_params=pltpu.CompilerParams(
            dimension_semantics=("parallel",),
            vmem_limit_bytes=vmem_limit,
        ),
    )(Xp, Wb)
    return out[:N] if n_pad != N else out


# speedup vs baseline: 1.3881x; 1.1057x over previous
import jax
import jax.numpy as jnp
from jax.experimental import pallas as pl
from jax.experimental.pallas import tpu as pltpu


def _round_up(x, m):
    return ((x + m - 1) // m) * m


def _matmul_kernel(x_ref, w_ref, o_ref):
    o_ref[...] = jnp.dot(
        x_ref[...].astype(jnp.bfloat16),
        w_ref[...],
        preferred_element_type=jnp.float32,
    ).astype(o_ref.dtype)


def kernel(X, W):
    N, D = X.shape
    D2, H = W.shape
    assert D == D2
    out_dtype = X.dtype

    Wb = W.astype(jnp.bfloat16)

    tm = min(2048, _round_up(N, 8))
    n_pad = _round_up(N, tm)
    Xp = X if n_pad == N else jnp.pad(X, ((0, n_pad - N), (0, 0)))
    grid = (n_pad // tm,)

    vmem_limit = min(
        2 * tm * D * 4 + 2 * tm * H * 4 + D * H * 2 + (4 << 20),
        64 * 1024 * 1024,
    )

    out = pl.pallas_call(
        _matmul_kernel,
        out_shape=jax.ShapeDtypeStruct((n_pad, H), out_dtype),
        grid=grid,
        in_specs=[
            pl.BlockSpec((tm, D), lambda i: (i, 0)),
            pl.BlockSpec((D, H), lambda i: (0, 0)),
        ],
        out_specs=pl.BlockSpec((tm, H), lambda i: (i, 0)),
        compiler_params=pltpu.CompilerParams(
            dimension_semantics=("parallel",),
            vmem_limit_bytes=vmem_limit,
        ),
    )(Xp, Wb)
    return out[:N] if n_pad != N else out
